# baseline (speedup 1.0000x reference)
import jax
import jax.numpy as jnp
from jax import lax
from jax.experimental import pallas as pl
from jax.experimental.pallas import tpu as pltpu
from jax.experimental.pallas import tpu_sc as plsc

N = 10000
D = 256
H = 256
C = 47
E = 160000

NCORE = 2
NSUB = 16
CHUNK = 128
EP = 163840
EPT_COL = EP // NSUB
NCH_COL = EPT_COL // CHUNK
EPT_EDGE = EP // (NSUB * NCORE)
NCH_EDGE = EPT_EDGE // CHUNK
ACC_R = 10240
ZROWS = ACC_R // NSUB
R_BLK = 5000
EPS = 1e-5


def _sc_copy_out(sid, src_ref, mk_dst, stripe=632):
  @pl.when(sid < NSUB - 1)
  def _():
    sl = pl.ds(sid * stripe, stripe)
    pltpu.sync_copy(src_ref.at[sl], mk_dst(sl))

  @pl.when(sid == NSUB - 1)
  def _():
    sl = pl.ds((NSUB - 1) * stripe, N - (NSUB - 1) * stripe)
    pltpu.sync_copy(src_ref.at[sl], mk_dst(sl))


def _sc_segment_sum(p_lo, p_hi, src2d, dst2d, zeros_w, *, edge_split):
  NCH_E0, NCH_E1 = 32, 48
  nch = max(NCH_E0, NCH_E1) if edge_split else NCH_COL
  nhalf = 1 if edge_split else 2
  nch2 = nch // nhalf
  stripe = 632
  mesh = plsc.VectorSubcoreMesh(core_axis_name="c", subcore_axis_name="s")
  out_type = [jax.ShapeDtypeStruct((NCORE, N, 128), jnp.float32)]
  scratch = [
      pltpu.VMEM((nch2, CHUNK), jnp.int32),
      pltpu.VMEM((nch2, CHUNK), jnp.int32),
      pltpu.VMEM((CHUNK, 128), jnp.float32),
      pltpu.VMEM((CHUNK, 128), jnp.float32),
      pltpu.VMEM_SHARED((ACC_R, 128), jnp.float32),
      pltpu.SemaphoreType.DMA,
      pltpu.SemaphoreType.DMA,
  ]

  def body(plo_hbm, phi_hbm, src_hbm, dst_hbm, zw_hbm, agg_hbm, sidx, didx,
           rows0, rows1, acc, sem0, sem1):
    core = lax.axis_index("c")
    sid = lax.axis_index("s")

    pltpu.sync_copy(zw_hbm, acc.at[pl.ds(sid * ZROWS, ZROWS)])
    plsc.subcore_barrier()

    def run(tbl, crow0, my_nch):
      for h in range(nhalf if my_nch == nch2 * nhalf else 1):
        rows_here = my_nch if my_nch < nch2 * nhalf else nch2
        pltpu.sync_copy(src_hbm.at[pl.ds(crow0 + h * rows_here, rows_here)],
                        sidx.at[pl.ds(0, rows_here)])
        pltpu.sync_copy(dst_hbm.at[pl.ds(crow0 + h * rows_here, rows_here)],
                        didx.at[pl.ds(0, rows_here)])
        pltpu.async_copy(tbl.at[sidx.at[0]], rows0, sem0)

        def step(k, carry):
          ca = 2 * k
          cb = 2 * k + 1
          pltpu.async_copy(tbl.at[sidx.at[cb]], rows1, sem1)
          pltpu.make_async_copy(tbl.at[sidx.at[ca]], rows0, sem0).wait()
          pltpu.sync_copy(rows0, acc.at[didx.at[ca]], add=True)

          @pl.when(k < rows_here // 2 - 1)
          def _():
            pltpu.async_copy(tbl.at[sidx.at[ca + 2]], rows0, sem0)
          pltpu.make_async_copy(tbl.at[sidx.at[cb]], rows1, sem1).wait()
          pltpu.sync_copy(rows1, acc.at[didx.at[cb]], add=True)
          return carry
        lax.fori_loop(0, rows_here // 2, step, 0)

    if edge_split:
      @pl.when(core == 0)
      def _():
        run(plo_hbm, sid * NCH_E0, NCH_E0)

      @pl.when(core == 1)
      def _():
        run(plo_hbm, NSUB * NCH_E0 + sid * NCH_E1, NCH_E1)
    else:
      @pl.when(core == 0)
      def _():
        run(plo_hbm, sid * NCH_COL, NCH_COL)

      @pl.when(core == 1)
      def _():
        run(phi_hbm, sid * NCH_COL, NCH_COL)

    plsc.subcore_barrier()

    @pl.when(core == 0)
    def _():
      _sc_copy_out(sid, acc, lambda sl: agg_hbm.at[0, sl], stripe)

    @pl.when(core == 1)
    def _():
      _sc_copy_out(sid, acc, lambda sl: agg_hbm.at[1, sl], stripe)

  fn = pl.kernel(body, mesh=mesh, out_type=out_type, scratch_types=scratch)
  res = fn(p_lo, p_hi, src2d, dst2d, zeros_w)
  return res[0] if isinstance(res, (list, tuple)) else res


def _sc_degree(dst2d, ones, zeros_w):
  mesh = plsc.VectorSubcoreMesh(core_axis_name="c", subcore_axis_name="s")
  out_type = [jax.ShapeDtypeStruct((NCORE, N, 128), jnp.float32)]
  scratch = [
      pltpu.VMEM((NCH_EDGE, CHUNK), jnp.int32),
      pltpu.VMEM((CHUNK, 128), jnp.float32),
      pltpu.VMEM_SHARED((ACC_R, 128), jnp.float32),
  ]

  def body(dst_hbm, ones_hbm, zw_hbm, deg_hbm, didx, onev, dacc):
    core = lax.axis_index("c")
    sid = lax.axis_index("s")
    pltpu.sync_copy(zw_hbm, dacc.at[pl.ds(sid * ZROWS, ZROWS)])
    pltpu.sync_copy(ones_hbm, onev)
    crow0 = (core * NSUB + sid) * NCH_EDGE
    pltpu.sync_copy(dst_hbm.at[pl.ds(crow0, NCH_EDGE)], didx)
    plsc.subcore_barrier()

    def step(ci, carry):
      pltpu.sync_copy(onev, dacc.at[didx.at[ci]], add=True)
      return carry
    lax.fori_loop(0, NCH_EDGE, step, 0)

    plsc.subcore_barrier()

    @pl.when(core == 0)
    def _():
      _sc_copy_out(sid, dacc, lambda sl: deg_hbm.at[0, sl])

    @pl.when(core == 1)
    def _():
      _sc_copy_out(sid, dacc, lambda sl: deg_hbm.at[1, sl])

  fn = pl.kernel(body, mesh=mesh, out_type=out_type, scratch_types=scratch)
  res = fn(dst2d, ones, zeros_w)
  return res[0] if isinstance(res, (list, tuple)) else res


def _make_proj_body(split_out):
  def body(x_ref, w_ref, *o_refs):
    r = jnp.dot(x_ref[...], w_ref[...], preferred_element_type=jnp.float32)
    if split_out:
      o_refs[0][...] = r[:, :128]
      o_refs[1][...] = r[:, 128:]
    else:
      o_refs[0][...] = r
  return body


def _proj_out(Wout, split_out):
  if split_out:
    return ([pl.BlockSpec((R_BLK, 128), lambda i: (i, 0)),
             pl.BlockSpec((R_BLK, 128), lambda i: (i, 0))],
            [jax.ShapeDtypeStruct((N, 128), jnp.float32),
             jax.ShapeDtypeStruct((N, 128), jnp.float32)])
  return (pl.BlockSpec((R_BLK, Wout), lambda i: (i, 0)),
          jax.ShapeDtypeStruct((N, Wout), jnp.float32))


def _tc_proj(x, W, *, split_out):
  Win = x.shape[1]
  Wout = W.shape[1]
  o_spec, o_shape = _proj_out(Wout, split_out)
  return pl.pallas_call(
      _make_proj_body(split_out),
      grid=(N // R_BLK,),
      in_specs=[
          pl.BlockSpec((R_BLK, Win), lambda i: (i, 0)),
          pl.BlockSpec((Win, Wout), lambda i: (0, 0)),
      ],
      out_specs=o_spec,
      out_shape=o_shape,
  )(x, W)


def _bn_relu(z, st, g, be):
  mean = st[0:1, :] / N
  var = st[1:2, :] / N - mean * mean
  inv = lax.rsqrt(var + EPS)
  return jnp.maximum((z - mean) * inv * g + be, 0.0)


def _make_combine_body(deg_partials, with_bn):
  def body(h_ref, *refs):
    if with_bn:
      pst_ref, g_ref, be_ref, w_ref, a_ref, d_ref, b_ref = refs[:7]
      out_refs = refs[7:]
      h = _bn_relu(h_ref[...], pst_ref[...], g_ref[...], be_ref[...])
    else:
      w_ref, a_ref, d_ref, b_ref = refs[:4]
      out_refs = refs[4:]
      h = h_ref[...]
    if deg_partials:
      z_ref, st_ref, rd_ref = out_refs
      d = d_ref[0][:, 0:1] + d_ref[1][:, 0:1]
      rd = 1.0 / jnp.maximum(d, 1.0)
      rd_ref[...] = jnp.broadcast_to(rd, (rd.shape[0], 16))
    else:
      z_ref, st_ref = out_refs
      rd = d_ref[...][:, 0:1]
    i = pl.program_id(0)
    s = jnp.dot(h, w_ref[...], preferred_element_type=jnp.float32)
    agg = jnp.concatenate([a_ref[0], a_ref[1]], axis=1)
    z = s + agg * rd + b_ref[...]
    z_ref[...] = z

    @pl.when(i == 0)
    def _():
      st_ref[...] = jnp.zeros_like(st_ref)

    w = z.shape[1]
    contrib = jnp.concatenate([
        jnp.sum(z, axis=0, keepdims=True),
        jnp.sum(z * z, axis=0, keepdims=True),
        jnp.zeros((6, w), jnp.float32),
    ], axis=0)
    st_ref[...] += contrib
  return body


def _tc_combine_stats(h, bn, Ws, aggp, d, b, *, deg_partials):
  Win = h.shape[1]
  W = Ws.shape[1]
  grid = (N // R_BLK,)
  if deg_partials:
    d_spec = pl.BlockSpec((NCORE, R_BLK, 128), lambda i: (0, i, 0))
  else:
    d_spec = pl.BlockSpec((R_BLK, 16), lambda i: (i, 0))
  in_specs = [pl.BlockSpec((R_BLK, Win), lambda i: (i, 0))]
  args = [h]
  if bn is not None:
    pst, g, be = bn
    in_specs += [
        pl.BlockSpec((8, Win), lambda i: (0, 0)),
        pl.BlockSpec((1, Win), lambda i: (0, 0)),
        pl.BlockSpec((1, Win), lambda i: (0, 0)),
    ]
    args += [pst, g, be]
  in_specs += [
      pl.BlockSpec((Win, W), lambda i: (0, 0)),
      pl.BlockSpec((NCORE, R_BLK, W // 2), lambda i: (0, i, 0)),
      d_spec,
      pl.BlockSpec((1, W), lambda i: (0, 0)),
  ]
  args += [Ws, aggp, d, b]
  out_specs = [
      pl.BlockSpec((R_BLK, W), lambda i: (i, 0)),
      pl.BlockSpec((8, W), lambda i: (0, 0)),
  ]
  out_shape = [
      jax.ShapeDtypeStruct((N, W), jnp.float32),
      jax.ShapeDtypeStruct((8, W), jnp.float32),
  ]
  if deg_partials:
    out_specs.append(pl.BlockSpec((R_BLK, 16), lambda i: (i, 0)))
    out_shape.append(jax.ShapeDtypeStruct((N, 16), jnp.float32))
  return pl.pallas_call(
      _make_combine_body(deg_partials, bn is not None),
      grid=grid,
      in_specs=in_specs,
      out_specs=out_specs,
      out_shape=out_shape,
  )(*args)


def _make_bn_proj_body(split_out):
  def body(z_ref, st_ref, g_ref, be_ref, w_ref, *o_refs):
    h = _bn_relu(z_ref[...], st_ref[...], g_ref[...], be_ref[...])
    r = jnp.dot(h, w_ref[...], preferred_element_type=jnp.float32)
    if split_out:
      o_refs[0][...] = r[:, :128]
      o_refs[1][...] = r[:, 128:]
    else:
      o_refs[0][...] = r
  return body


def _tc_bn_proj(z, st, g, be, W, *, split_out):
  Win = z.shape[1]
  Wout = W.shape[1]
  o_spec, o_shape = _proj_out(Wout, split_out)
  return pl.pallas_call(
      _make_bn_proj_body(split_out),
      grid=(N // R_BLK,),
      in_specs=[
          pl.BlockSpec((R_BLK, Win), lambda i: (i, 0)),
          pl.BlockSpec((8, Win), lambda i: (0, 0)),
          pl.BlockSpec((1, Win), lambda i: (0, 0)),
          pl.BlockSpec((1, Win), lambda i: (0, 0)),
          pl.BlockSpec((Win, Wout), lambda i: (0, 0)),
      ],
      out_specs=o_spec,
      out_shape=o_shape,
  )(z, st, g, be, W)


def _k_final_body(z2_ref, pst_ref, g_ref, be_ref, w_ref, a_ref, rd_ref, b_ref,
                  o_ref):
  h = _bn_relu(z2_ref[...], pst_ref[...], g_ref[...], be_ref[...])
  s = jnp.dot(h, w_ref[...], preferred_element_type=jnp.float32)
  agg = a_ref[0] + a_ref[1]
  rd = rd_ref[...][:, 0:1]
  z = s + agg * rd + b_ref[...]
  col = lax.broadcasted_iota(jnp.int32, z.shape, 1)
  valid = col < C
  zm = jnp.where(valid, z, -jnp.inf)
  m = jnp.max(zm, axis=1, keepdims=True)
  ex = jnp.where(valid, jnp.exp(zm - m), 0.0)
  lse = jnp.log(jnp.sum(ex, axis=1, keepdims=True))
  o_ref[...] = zm - m - lse


def _tc_final(z2, st2, g, be, Ws, aggp, rdeg, b):
  Win = z2.shape[1]
  W = Ws.shape[1]
  grid = (N // R_BLK,)
  return pl.pallas_call(
      _k_final_body,
      grid=grid,
      in_specs=[
          pl.BlockSpec((R_BLK, Win), lambda i: (i, 0)),
          pl.BlockSpec((8, Win), lambda i: (0, 0)),
          pl.BlockSpec((1, Win), lambda i: (0, 0)),
          pl.BlockSpec((1, Win), lambda i: (0, 0)),
          pl.BlockSpec((Win, W), lambda i: (0, 0)),
          pl.BlockSpec((NCORE, R_BLK, W), lambda i: (0, i, 0)),
          pl.BlockSpec((R_BLK, 16), lambda i: (i, 0)),
          pl.BlockSpec((1, W), lambda i: (0, 0)),
      ],
      out_specs=pl.BlockSpec((R_BLK, W), lambda i: (i, 0)),
      out_shape=jax.ShapeDtypeStruct((N, W), jnp.float32),
  )(z2, st2, g, be, Ws, aggp, rdeg, b)


def kernel(x, edge_index, W_self1, W_neigh1, b1, gamma1, beta1,
           W_self2, W_neigh2, b2, gamma2, beta2,
           W_self3, W_neigh3, b3):
  src = jnp.concatenate(
      [edge_index[0], jnp.zeros((EP - E,), jnp.int32)]).reshape(-1, CHUNK)
  pad_dst = N + jnp.arange(EP - E, dtype=jnp.int32) % (ACC_R - N)
  dst = jnp.concatenate([edge_index[1], pad_dst]).reshape(-1, CHUNK)
  zw128 = jnp.zeros((ZROWS, 128), jnp.float32)
  ones = jnp.ones((CHUNK, 128), jnp.float32)

  b1r = b1.reshape(1, H)
  b2r = b2.reshape(1, H)
  g1 = gamma1.reshape(1, H)
  be1 = beta1.reshape(1, H)
  g2 = gamma2.reshape(1, H)
  be2 = beta2.reshape(1, H)
  Wn3 = jnp.pad(W_neigh3, ((0, 0), (0, 128 - C)))
  Ws3 = jnp.pad(W_self3, ((0, 0), (0, 128 - C)))
  b3r = jnp.pad(b3, (0, 128 - C)).reshape(1, 128)

  degp = _sc_degree(dst, ones, zw128)
  p1lo, p1hi = _tc_proj(x, W_neigh1, split_out=True)
  agg1 = _sc_segment_sum(p1lo, p1hi, src, dst, zw128, edge_split=False)
  z1, st1, rdeg = _tc_combine_stats(x, None, W_self1, agg1, degp, b1r,
                                    deg_partials=True)
  p2lo, p2hi = _tc_bn_proj(z1, st1, g1, be1, W_neigh2, split_out=True)
  agg2 = _sc_segment_sum(p2lo, p2hi, src, dst, zw128, edge_split=False)
  z2, st2 = _tc_combine_stats(z1, (st1, g1, be1), W_self2, agg2, rdeg, b2r,
                              deg_partials=False)
  p3 = _tc_bn_proj(z2, st2, g2, be2, Wn3, split_out=False)
  agg3 = _sc_segment_sum(p3, p3, src, dst, zw128, edge_split=True)
  o = _tc_final(z2, st2, g2, be2, Ws3, agg3, rdeg, b3r)
  return o[:, :C]


# speedup vs baseline: 1.0124x; 1.0124x over previous
import jax
import jax.numpy as jnp
from jax import lax
from jax.experimental import pallas as pl
from jax.experimental.pallas import tpu as pltpu
from jax.experimental.pallas import tpu_sc as plsc

N = 10000
D = 256
H = 256
C = 47
E = 160000

NCORE = 2
NSUB = 16
CHUNK = 128
EP = 163840
EPT_COL = EP // NSUB
NCH_COL = EPT_COL // CHUNK
EPT_EDGE = EP // (NSUB * NCORE)
NCH_EDGE = EPT_EDGE // CHUNK
ACC_R = 10240
ZROWS = ACC_R // NSUB
R_BLK = 5000
EPS = 1e-5


def _sc_copy_out(sid, src_ref, mk_dst, stripe=632):
  @pl.when(sid < NSUB - 1)
  def _():
    sl = pl.ds(sid * stripe, stripe)
    pltpu.sync_copy(src_ref.at[sl], mk_dst(sl))

  @pl.when(sid == NSUB - 1)
  def _():
    sl = pl.ds((NSUB - 1) * stripe, N - (NSUB - 1) * stripe)
    pltpu.sync_copy(src_ref.at[sl], mk_dst(sl))


def _sc_segment_sum(p_lo, p_hi, src2d, dst2d, zeros_w, *, edge_split):
  NCH_E0, NCH_E1 = 48, 32
  nch = max(NCH_E0, NCH_E1) if edge_split else NCH_COL
  nhalf = 1 if edge_split else 2
  nch2 = nch // nhalf
  stripe = 632
  mesh = plsc.VectorSubcoreMesh(core_axis_name="c", subcore_axis_name="s")
  out_type = [jax.ShapeDtypeStruct((NCORE, N, 128), jnp.float32)]
  scratch = [
      pltpu.VMEM((nch2, CHUNK), jnp.int32),
      pltpu.VMEM((nch2, CHUNK), jnp.int32),
      pltpu.VMEM((CHUNK, 128), jnp.float32),
      pltpu.VMEM((CHUNK, 128), jnp.float32),
      pltpu.VMEM_SHARED((ACC_R, 128), jnp.float32),
      pltpu.SemaphoreType.DMA,
      pltpu.SemaphoreType.DMA,
  ]

  def body(plo_hbm, phi_hbm, src_hbm, dst_hbm, zw_hbm, agg_hbm, sidx, didx,
           rows0, rows1, acc, sem0, sem1):
    core = lax.axis_index("c")
    sid = lax.axis_index("s")

    pltpu.sync_copy(zw_hbm, acc.at[pl.ds(sid * ZROWS, ZROWS)])
    plsc.subcore_barrier()

    def run(tbl, crow0, my_nch):
      for h in range(nhalf if my_nch == nch2 * nhalf else 1):
        rows_here = my_nch if my_nch < nch2 * nhalf else nch2
        pltpu.sync_copy(src_hbm.at[pl.ds(crow0 + h * rows_here, rows_here)],
                        sidx.at[pl.ds(0, rows_here)])
        pltpu.sync_copy(dst_hbm.at[pl.ds(crow0 + h * rows_here, rows_here)],
                        didx.at[pl.ds(0, rows_here)])
        pltpu.async_copy(tbl.at[sidx.at[0]], rows0, sem0)

        def step(k, carry):
          ca = 2 * k
          cb = 2 * k + 1
          pltpu.async_copy(tbl.at[sidx.at[cb]], rows1, sem1)
          pltpu.make_async_copy(tbl.at[sidx.at[ca]], rows0, sem0).wait()
          pltpu.sync_copy(rows0, acc.at[didx.at[ca]], add=True)

          @pl.when(k < rows_here // 2 - 1)
          def _():
            pltpu.async_copy(tbl.at[sidx.at[ca + 2]], rows0, sem0)
          pltpu.make_async_copy(tbl.at[sidx.at[cb]], rows1, sem1).wait()
          pltpu.sync_copy(rows1, acc.at[didx.at[cb]], add=True)
          return carry
        lax.fori_loop(0, rows_here // 2, step, 0)

    if edge_split:
      @pl.when(core == 0)
      def _():
        run(plo_hbm, sid * NCH_E0, NCH_E0)

      @pl.when(core == 1)
      def _():
        run(plo_hbm, NSUB * NCH_E0 + sid * NCH_E1, NCH_E1)
    else:
      @pl.when(core == 0)
      def _():
        run(plo_hbm, sid * NCH_COL, NCH_COL)

      @pl.when(core == 1)
      def _():
        run(phi_hbm, sid * NCH_COL, NCH_COL)

    plsc.subcore_barrier()

    @pl.when(core == 0)
    def _():
      _sc_copy_out(sid, acc, lambda sl: agg_hbm.at[0, sl], stripe)

    @pl.when(core == 1)
    def _():
      _sc_copy_out(sid, acc, lambda sl: agg_hbm.at[1, sl], stripe)

  fn = pl.kernel(body, mesh=mesh, out_type=out_type, scratch_types=scratch)
  res = fn(p_lo, p_hi, src2d, dst2d, zeros_w)
  return res[0] if isinstance(res, (list, tuple)) else res


def _sc_degree(dst2d, ones, zeros_w):
  mesh = plsc.VectorSubcoreMesh(core_axis_name="c", subcore_axis_name="s")
  out_type = [jax.ShapeDtypeStruct((NCORE, N, 128), jnp.float32)]
  scratch = [
      pltpu.VMEM((NCH_EDGE, CHUNK), jnp.int32),
      pltpu.VMEM((CHUNK, 128), jnp.float32),
      pltpu.VMEM_SHARED((ACC_R, 128), jnp.float32),
  ]

  def body(dst_hbm, ones_hbm, zw_hbm, deg_hbm, didx, onev, dacc):
    core = lax.axis_index("c")
    sid = lax.axis_index("s")
    pltpu.sync_copy(zw_hbm, dacc.at[pl.ds(sid * ZROWS, ZROWS)])
    pltpu.sync_copy(ones_hbm, onev)
    crow0 = (core * NSUB + sid) * NCH_EDGE
    pltpu.sync_copy(dst_hbm.at[pl.ds(crow0, NCH_EDGE)], didx)
    plsc.subcore_barrier()

    def step(ci, carry):
      pltpu.sync_copy(onev, dacc.at[didx.at[ci]], add=True)
      return carry
    lax.fori_loop(0, NCH_EDGE, step, 0)

    plsc.subcore_barrier()

    @pl.when(core == 0)
    def _():
      _sc_copy_out(sid, dacc, lambda sl: deg_hbm.at[0, sl])

    @pl.when(core == 1)
    def _():
      _sc_copy_out(sid, dacc, lambda sl: deg_hbm.at[1, sl])

  fn = pl.kernel(body, mesh=mesh, out_type=out_type, scratch_types=scratch)
  res = fn(dst2d, ones, zeros_w)
  return res[0] if isinstance(res, (list, tuple)) else res


def _make_proj_body(split_out):
  def body(x_ref, w_ref, *o_refs):
    r = jnp.dot(x_ref[...], w_ref[...], preferred_element_type=jnp.float32)
    if split_out:
      o_refs[0][...] = r[:, :128]
      o_refs[1][...] = r[:, 128:]
    else:
      o_refs[0][...] = r
  return body


def _proj_out(Wout, split_out):
  if split_out:
    return ([pl.BlockSpec((R_BLK, 128), lambda i: (i, 0)),
             pl.BlockSpec((R_BLK, 128), lambda i: (i, 0))],
            [jax.ShapeDtypeStruct((N, 128), jnp.float32),
             jax.ShapeDtypeStruct((N, 128), jnp.float32)])
  return (pl.BlockSpec((R_BLK, Wout), lambda i: (i, 0)),
          jax.ShapeDtypeStruct((N, Wout), jnp.float32))


def _tc_proj(x, W, *, split_out):
  Win = x.shape[1]
  Wout = W.shape[1]
  o_spec, o_shape = _proj_out(Wout, split_out)
  return pl.pallas_call(
      _make_proj_body(split_out),
      grid=(N // R_BLK,),
      in_specs=[
          pl.BlockSpec((R_BLK, Win), lambda i: (i, 0)),
          pl.BlockSpec((Win, Wout), lambda i: (0, 0)),
      ],
      out_specs=o_spec,
      out_shape=o_shape,
  )(x, W)


def _bn_relu(z, st, g, be):
  mean = st[0:1, :] / N
  var = st[1:2, :] / N - mean * mean
  inv = lax.rsqrt(var + EPS)
  return jnp.maximum((z - mean) * inv * g + be, 0.0)


def _make_combine_body(deg_partials, with_bn):
  def body(h_ref, *refs):
    if with_bn:
      pst_ref, g_ref, be_ref, w_ref, a_ref, d_ref, b_ref = refs[:7]
      out_refs = refs[7:]
      h = _bn_relu(h_ref[...], pst_ref[...], g_ref[...], be_ref[...])
    else:
      w_ref, a_ref, d_ref, b_ref = refs[:4]
      out_refs = refs[4:]
      h = h_ref[...]
    if deg_partials:
      z_ref, st_ref, rd_ref = out_refs
      d = d_ref[0][:, 0:1] + d_ref[1][:, 0:1]
      rd = 1.0 / jnp.maximum(d, 1.0)
      rd_ref[...] = jnp.broadcast_to(rd, (rd.shape[0], 16))
    else:
      z_ref, st_ref = out_refs
      rd = d_ref[...][:, 0:1]
    i = pl.program_id(0)
    s = jnp.dot(h, w_ref[...], preferred_element_type=jnp.float32)
    agg = jnp.concatenate([a_ref[0], a_ref[1]], axis=1)
    z = s + agg * rd + b_ref[...]
    z_ref[...] = z

    @pl.when(i == 0)
    def _():
      st_ref[...] = jnp.zeros_like(st_ref)

    w = z.shape[1]
    contrib = jnp.concatenate([
        jnp.sum(z, axis=0, keepdims=True),
        jnp.sum(z * z, axis=0, keepdims=True),
        jnp.zeros((6, w), jnp.float32),
    ], axis=0)
    st_ref[...] += contrib
  return body


def _tc_combine_stats(h, bn, Ws, aggp, d, b, *, deg_partials):
  Win = h.shape[1]
  W = Ws.shape[1]
  grid = (N // R_BLK,)
  if deg_partials:
    d_spec = pl.BlockSpec((NCORE, R_BLK, 128), lambda i: (0, i, 0))
  else:
    d_spec = pl.BlockSpec((R_BLK, 16), lambda i: (i, 0))
  in_specs = [pl.BlockSpec((R_BLK, Win), lambda i: (i, 0))]
  args = [h]
  if bn is not None:
    pst, g, be = bn
    in_specs += [
        pl.BlockSpec((8, Win), lambda i: (0, 0)),
        pl.BlockSpec((1, Win), lambda i: (0, 0)),
        pl.BlockSpec((1, Win), lambda i: (0, 0)),
    ]
    args += [pst, g, be]
  in_specs += [
      pl.BlockSpec((Win, W), lambda i: (0, 0)),
      pl.BlockSpec((NCORE, R_BLK, W // 2), lambda i: (0, i, 0)),
      d_spec,
      pl.BlockSpec((1, W), lambda i: (0, 0)),
  ]
  args += [Ws, aggp, d, b]
  out_specs = [
      pl.BlockSpec((R_BLK, W), lambda i: (i, 0)),
      pl.BlockSpec((8, W), lambda i: (0, 0)),
  ]
  out_shape = [
      jax.ShapeDtypeStruct((N, W), jnp.float32),
      jax.ShapeDtypeStruct((8, W), jnp.float32),
  ]
  if deg_partials:
    out_specs.append(pl.BlockSpec((R_BLK, 16), lambda i: (i, 0)))
    out_shape.append(jax.ShapeDtypeStruct((N, 16), jnp.float32))
  return pl.pallas_call(
      _make_combine_body(deg_partials, bn is not None),
      grid=grid,
      in_specs=in_specs,
      out_specs=out_specs,
      out_shape=out_shape,
  )(*args)


def _make_bn_proj_body(split_out):
  def body(z_ref, st_ref, g_ref, be_ref, w_ref, *o_refs):
    h = _bn_relu(z_ref[...], st_ref[...], g_ref[...], be_ref[...])
    r = jnp.dot(h, w_ref[...], preferred_element_type=jnp.float32)
    if split_out:
      o_refs[0][...] = r[:, :128]
      o_refs[1][...] = r[:, 128:]
    else:
      o_refs[0][...] = r
  return body


def _tc_bn_proj(z, st, g, be, W, *, split_out):
  Win = z.shape[1]
  Wout = W.shape[1]
  o_spec, o_shape = _proj_out(Wout, split_out)
  return pl.pallas_call(
      _make_bn_proj_body(split_out),
      grid=(N // R_BLK,),
      in_specs=[
          pl.BlockSpec((R_BLK, Win), lambda i: (i, 0)),
          pl.BlockSpec((8, Win), lambda i: (0, 0)),
          pl.BlockSpec((1, Win), lambda i: (0, 0)),
          pl.BlockSpec((1, Win), lambda i: (0, 0)),
          pl.BlockSpec((Win, Wout), lambda i: (0, 0)),
      ],
      out_specs=o_spec,
      out_shape=o_shape,
  )(z, st, g, be, W)


def _k_final_body(z2_ref, pst_ref, g_ref, be_ref, w_ref, a_ref, rd_ref, b_ref,
                  o_ref):
  h = _bn_relu(z2_ref[...], pst_ref[...], g_ref[...], be_ref[...])
  s = jnp.dot(h, w_ref[...], preferred_element_type=jnp.float32)
  agg = a_ref[0] + a_ref[1]
  rd = rd_ref[...][:, 0:1]
  z = s + agg * rd + b_ref[...]
  col = lax.broadcasted_iota(jnp.int32, z.shape, 1)
  valid = col < C
  zm = jnp.where(valid, z, -jnp.inf)
  m = jnp.max(zm, axis=1, keepdims=True)
  ex = jnp.where(valid, jnp.exp(zm - m), 0.0)
  lse = jnp.log(jnp.sum(ex, axis=1, keepdims=True))
  o_ref[...] = zm - m - lse


def _tc_final(z2, st2, g, be, Ws, aggp, rdeg, b):
  Win = z2.shape[1]
  W = Ws.shape[1]
  grid = (N // R_BLK,)
  return pl.pallas_call(
      _k_final_body,
      grid=grid,
      in_specs=[
          pl.BlockSpec((R_BLK, Win), lambda i: (i, 0)),
          pl.BlockSpec((8, Win), lambda i: (0, 0)),
          pl.BlockSpec((1, Win), lambda i: (0, 0)),
          pl.BlockSpec((1, Win), lambda i: (0, 0)),
          pl.BlockSpec((Win, W), lambda i: (0, 0)),
          pl.BlockSpec((NCORE, R_BLK, W), lambda i: (0, i, 0)),
          pl.BlockSpec((R_BLK, 16), lambda i: (i, 0)),
          pl.BlockSpec((1, W), lambda i: (0, 0)),
      ],
      out_specs=pl.BlockSpec((R_BLK, W), lambda i: (i, 0)),
      out_shape=jax.ShapeDtypeStruct((N, W), jnp.float32),
  )(z2, st2, g, be, Ws, aggp, rdeg, b)


def kernel(x, edge_index, W_self1, W_neigh1, b1, gamma1, beta1,
           W_self2, W_neigh2, b2, gamma2, beta2,
           W_self3, W_neigh3, b3):
  src = jnp.concatenate(
      [edge_index[0], jnp.zeros((EP - E,), jnp.int32)]).reshape(-1, CHUNK)
  pad_dst = N + jnp.arange(EP - E, dtype=jnp.int32) % (ACC_R - N)
  dst = jnp.concatenate([edge_index[1], pad_dst]).reshape(-1, CHUNK)
  zw128 = jnp.zeros((ZROWS, 128), jnp.float32)
  ones = jnp.ones((CHUNK, 128), jnp.float32)

  b1r = b1.reshape(1, H)
  b2r = b2.reshape(1, H)
  g1 = gamma1.reshape(1, H)
  be1 = beta1.reshape(1, H)
  g2 = gamma2.reshape(1, H)
  be2 = beta2.reshape(1, H)
  Wn3 = jnp.pad(W_neigh3, ((0, 0), (0, 128 - C)))
  Ws3 = jnp.pad(W_self3, ((0, 0), (0, 128 - C)))
  b3r = jnp.pad(b3, (0, 128 - C)).reshape(1, 128)

  degp = _sc_degree(dst, ones, zw128)
  p1lo, p1hi = _tc_proj(x, W_neigh1, split_out=True)
  agg1 = _sc_segment_sum(p1lo, p1hi, src, dst, zw128, edge_split=False)
  z1, st1, rdeg = _tc_combine_stats(x, None, W_self1, agg1, degp, b1r,
                                    deg_partials=True)
  p2lo, p2hi = _tc_bn_proj(z1, st1, g1, be1, W_neigh2, split_out=True)
  agg2 = _sc_segment_sum(p2lo, p2hi, src, dst, zw128, edge_split=False)
  z2, st2 = _tc_combine_stats(z1, (st1, g1, be1), W_self2, agg2, rdeg, b2r,
                              deg_partials=False)
  p3 = _tc_bn_proj(z2, st2, g2, be2, Wn3, split_out=False)
  agg3 = _sc_segment_sum(p3, p3, src, dst, zw128, edge_split=True)
  o = _tc_final(z2, st2, g2, be2, Ws3, agg3, rdeg, b3r)
  return o[:, :C]


# speedup vs baseline: 1.0179x; 1.0055x over previous
import jax
import jax.numpy as jnp
from jax import lax
from jax.experimental import pallas as pl
from jax.experimental.pallas import tpu as pltpu
from jax.experimental.pallas import tpu_sc as plsc

N = 10000
D = 256
H = 256
C = 47
E = 160000

NCORE = 2
NSUB = 16
CHUNK = 128
EP = 163840
EPT_COL = EP // NSUB
NCH_COL = EPT_COL // CHUNK
EPT_EDGE = EP // (NSUB * NCORE)
NCH_EDGE = EPT_EDGE // CHUNK
ACC_R = 10240
ZROWS = ACC_R // NSUB
R_BLK = 5000
EPS = 1e-5


def _sc_copy_out(sid, src_ref, mk_dst, stripe=632):
  @pl.when(sid < NSUB - 1)
  def _():
    sl = pl.ds(sid * stripe, stripe)
    pltpu.sync_copy(src_ref.at[sl], mk_dst(sl))

  @pl.when(sid == NSUB - 1)
  def _():
    sl = pl.ds((NSUB - 1) * stripe, N - (NSUB - 1) * stripe)
    pltpu.sync_copy(src_ref.at[sl], mk_dst(sl))


def _sc_segment_sum(p_lo, p_hi, src2d, dst2d, zeros_w, *, edge_split):
  NCH_E0, NCH_E1 = 56, 24
  nch = max(NCH_E0, NCH_E1) if edge_split else NCH_COL
  nhalf = 1 if edge_split else 2
  nch2 = nch // nhalf
  stripe = 632
  mesh = plsc.VectorSubcoreMesh(core_axis_name="c", subcore_axis_name="s")
  out_type = [jax.ShapeDtypeStruct((NCORE, N, 128), jnp.float32)]
  scratch = [
      pltpu.VMEM((nch2, CHUNK), jnp.int32),
      pltpu.VMEM((nch2, CHUNK), jnp.int32),
      pltpu.VMEM((CHUNK, 128), jnp.float32),
      pltpu.VMEM((CHUNK, 128), jnp.float32),
      pltpu.VMEM_SHARED((ACC_R, 128), jnp.float32),
      pltpu.SemaphoreType.DMA,
      pltpu.SemaphoreType.DMA,
  ]

  def body(plo_hbm, phi_hbm, src_hbm, dst_hbm, zw_hbm, agg_hbm, sidx, didx,
           rows0, rows1, acc, sem0, sem1):
    core = lax.axis_index("c")
    sid = lax.axis_index("s")

    pltpu.sync_copy(zw_hbm, acc.at[pl.ds(sid * ZROWS, ZROWS)])
    plsc.subcore_barrier()

    def run(tbl, crow0, my_nch):
      for h in range(nhalf if my_nch == nch2 * nhalf else 1):
        rows_here = my_nch if my_nch < nch2 * nhalf else nch2
        pltpu.sync_copy(src_hbm.at[pl.ds(crow0 + h * rows_here, rows_here)],
                        sidx.at[pl.ds(0, rows_here)])
        pltpu.sync_copy(dst_hbm.at[pl.ds(crow0 + h * rows_here, rows_here)],
                        didx.at[pl.ds(0, rows_here)])
        pltpu.async_copy(tbl.at[sidx.at[0]], rows0, sem0)

        def step(k, carry):
          ca = 2 * k
          cb = 2 * k + 1
          pltpu.async_copy(tbl.at[sidx.at[cb]], rows1, sem1)
          pltpu.make_async_copy(tbl.at[sidx.at[ca]], rows0, sem0).wait()
          pltpu.sync_copy(rows0, acc.at[didx.at[ca]], add=True)

          @pl.when(k < rows_here // 2 - 1)
          def _():
            pltpu.async_copy(tbl.at[sidx.at[ca + 2]], rows0, sem0)
          pltpu.make_async_copy(tbl.at[sidx.at[cb]], rows1, sem1).wait()
          pltpu.sync_copy(rows1, acc.at[didx.at[cb]], add=True)
          return carry
        lax.fori_loop(0, rows_here // 2, step, 0)

    if edge_split:
      @pl.when(core == 0)
      def _():
        run(plo_hbm, sid * NCH_E0, NCH_E0)

      @pl.when(core == 1)
      def _():
        run(plo_hbm, NSUB * NCH_E0 + sid * NCH_E1, NCH_E1)
    else:
      @pl.when(core == 0)
      def _():
        run(plo_hbm, sid * NCH_COL, NCH_COL)

      @pl.when(core == 1)
      def _():
        run(phi_hbm, sid * NCH_COL, NCH_COL)

    plsc.subcore_barrier()

    @pl.when(core == 0)
    def _():
      _sc_copy_out(sid, acc, lambda sl: agg_hbm.at[0, sl], stripe)

    @pl.when(core == 1)
    def _():
      _sc_copy_out(sid, acc, lambda sl: agg_hbm.at[1, sl], stripe)

  fn = pl.kernel(body, mesh=mesh, out_type=out_type, scratch_types=scratch)
  res = fn(p_lo, p_hi, src2d, dst2d, zeros_w)
  return res[0] if isinstance(res, (list, tuple)) else res


def _sc_degree(dst2d, ones, zeros_w):
  mesh = plsc.VectorSubcoreMesh(core_axis_name="c", subcore_axis_name="s")
  out_type = [jax.ShapeDtypeStruct((NCORE, N, 128), jnp.float32)]
  scratch = [
      pltpu.VMEM((NCH_EDGE, CHUNK), jnp.int32),
      pltpu.VMEM((CHUNK, 128), jnp.float32),
      pltpu.VMEM_SHARED((ACC_R, 128), jnp.float32),
  ]

  def body(dst_hbm, ones_hbm, zw_hbm, deg_hbm, didx, onev, dacc):
    core = lax.axis_index("c")
    sid = lax.axis_index("s")
    pltpu.sync_copy(zw_hbm, dacc.at[pl.ds(sid * ZROWS, ZROWS)])
    pltpu.sync_copy(ones_hbm, onev)
    crow0 = (core * NSUB + sid) * NCH_EDGE
    pltpu.sync_copy(dst_hbm.at[pl.ds(crow0, NCH_EDGE)], didx)
    plsc.subcore_barrier()

    def step(ci, carry):
      pltpu.sync_copy(onev, dacc.at[didx.at[ci]], add=True)
      return carry
    lax.fori_loop(0, NCH_EDGE, step, 0)

    plsc.subcore_barrier()

    @pl.when(core == 0)
    def _():
      _sc_copy_out(sid, dacc, lambda sl: deg_hbm.at[0, sl])

    @pl.when(core == 1)
    def _():
      _sc_copy_out(sid, dacc, lambda sl: deg_hbm.at[1, sl])

  fn = pl.kernel(body, mesh=mesh, out_type=out_type, scratch_types=scratch)
  res = fn(dst2d, ones, zeros_w)
  return res[0] if isinstance(res, (list, tuple)) else res


def _make_proj_body(split_out):
  def body(x_ref, w_ref, *o_refs):
    r = jnp.dot(x_ref[...], w_ref[...], preferred_element_type=jnp.float32)
    if split_out:
      o_refs[0][...] = r[:, :128]
      o_refs[1][...] = r[:, 128:]
    else:
      o_refs[0][...] = r
  return body


def _proj_out(Wout, split_out):
  if split_out:
    return ([pl.BlockSpec((R_BLK, 128), lambda i: (i, 0)),
             pl.BlockSpec((R_BLK, 128), lambda i: (i, 0))],
            [jax.ShapeDtypeStruct((N, 128), jnp.float32),
             jax.ShapeDtypeStruct((N, 128), jnp.float32)])
  return (pl.BlockSpec((R_BLK, Wout), lambda i: (i, 0)),
          jax.ShapeDtypeStruct((N, Wout), jnp.float32))


def _tc_proj(x, W, *, split_out):
  Win = x.shape[1]
  Wout = W.shape[1]
  o_spec, o_shape = _proj_out(Wout, split_out)
  return pl.pallas_call(
      _make_proj_body(split_out),
      grid=(N // R_BLK,),
      in_specs=[
          pl.BlockSpec((R_BLK, Win), lambda i: (i, 0)),
          pl.BlockSpec((Win, Wout), lambda i: (0, 0)),
      ],
      out_specs=o_spec,
      out_shape=o_shape,
  )(x, W)


def _bn_relu(z, st, g, be):
  mean = st[0:1, :] / N
  var = st[1:2, :] / N - mean * mean
  inv = lax.rsqrt(var + EPS)
  return jnp.maximum((z - mean) * inv * g + be, 0.0)


def _make_combine_body(deg_partials, with_bn):
  def body(h_ref, *refs):
    if with_bn:
      pst_ref, g_ref, be_ref, w_ref, a_ref, d_ref, b_ref = refs[:7]
      out_refs = refs[7:]
      h = _bn_relu(h_ref[...], pst_ref[...], g_ref[...], be_ref[...])
    else:
      w_ref, a_ref, d_ref, b_ref = refs[:4]
      out_refs = refs[4:]
      h = h_ref[...]
    if deg_partials:
      z_ref, st_ref, rd_ref = out_refs
      d = d_ref[0][:, 0:1] + d_ref[1][:, 0:1]
      rd = 1.0 / jnp.maximum(d, 1.0)
      rd_ref[...] = jnp.broadcast_to(rd, (rd.shape[0], 16))
    else:
      z_ref, st_ref = out_refs
      rd = d_ref[...][:, 0:1]
    i = pl.program_id(0)
    s = jnp.dot(h, w_ref[...], preferred_element_type=jnp.float32)
    agg = jnp.concatenate([a_ref[0], a_ref[1]], axis=1)
    z = s + agg * rd + b_ref[...]
    z_ref[...] = z

    @pl.when(i == 0)
    def _():
      st_ref[...] = jnp.zeros_like(st_ref)

    w = z.shape[1]
    contrib = jnp.concatenate([
        jnp.sum(z, axis=0, keepdims=True),
        jnp.sum(z * z, axis=0, keepdims=True),
        jnp.zeros((6, w), jnp.float32),
    ], axis=0)
    st_ref[...] += contrib
  return body


def _tc_combine_stats(h, bn, Ws, aggp, d, b, *, deg_partials):
  Win = h.shape[1]
  W = Ws.shape[1]
  grid = (N // R_BLK,)
  if deg_partials:
    d_spec = pl.BlockSpec((NCORE, R_BLK, 128), lambda i: (0, i, 0))
  else:
    d_spec = pl.BlockSpec((R_BLK, 16), lambda i: (i, 0))
  in_specs = [pl.BlockSpec((R_BLK, Win), lambda i: (i, 0))]
  args = [h]
  if bn is not None:
    pst, g, be = bn
    in_specs += [
        pl.BlockSpec((8, Win), lambda i: (0, 0)),
        pl.BlockSpec((1, Win), lambda i: (0, 0)),
        pl.BlockSpec((1, Win), lambda i: (0, 0)),
    ]
    args += [pst, g, be]
  in_specs += [
      pl.BlockSpec((Win, W), lambda i: (0, 0)),
      pl.BlockSpec((NCORE, R_BLK, W // 2), lambda i: (0, i, 0)),
      d_spec,
      pl.BlockSpec((1, W), lambda i: (0, 0)),
  ]
  args += [Ws, aggp, d, b]
  out_specs = [
      pl.BlockSpec((R_BLK, W), lambda i: (i, 0)),
      pl.BlockSpec((8, W), lambda i: (0, 0)),
  ]
  out_shape = [
      jax.ShapeDtypeStruct((N, W), jnp.float32),
      jax.ShapeDtypeStruct((8, W), jnp.float32),
  ]
  if deg_partials:
    out_specs.append(pl.BlockSpec((R_BLK, 16), lambda i: (i, 0)))
    out_shape.append(jax.ShapeDtypeStruct((N, 16), jnp.float32))
  return pl.pallas_call(
      _make_combine_body(deg_partials, bn is not None),
      grid=grid,
      in_specs=in_specs,
      out_specs=out_specs,
      out_shape=out_shape,
  )(*args)


def _make_bn_proj_body(split_out):
  def body(z_ref, st_ref, g_ref, be_ref, w_ref, *o_refs):
    h = _bn_relu(z_ref[...], st_ref[...], g_ref[...], be_ref[...])
    r = jnp.dot(h, w_ref[...], preferred_element_type=jnp.float32)
    if split_out:
      o_refs[0][...] = r[:, :128]
      o_refs[1][...] = r[:, 128:]
    else:
      o_refs[0][...] = r
  return body


def _tc_bn_proj(z, st, g, be, W, *, split_out):
  Win = z.shape[1]
  Wout = W.shape[1]
  o_spec, o_shape = _proj_out(Wout, split_out)
  return pl.pallas_call(
      _make_bn_proj_body(split_out),
      grid=(N // R_BLK,),
      in_specs=[
          pl.BlockSpec((R_BLK, Win), lambda i: (i, 0)),
          pl.BlockSpec((8, Win), lambda i: (0, 0)),
          pl.BlockSpec((1, Win), lambda i: (0, 0)),
          pl.BlockSpec((1, Win), lambda i: (0, 0)),
          pl.BlockSpec((Win, Wout), lambda i: (0, 0)),
      ],
      out_specs=o_spec,
      out_shape=o_shape,
  )(z, st, g, be, W)


def _k_final_body(z2_ref, pst_ref, g_ref, be_ref, w_ref, a_ref, rd_ref, b_ref,
                  o_ref):
  h = _bn_relu(z2_ref[...], pst_ref[...], g_ref[...], be_ref[...])
  s = jnp.dot(h, w_ref[...], preferred_element_type=jnp.float32)
  agg = a_ref[0] + a_ref[1]
  rd = rd_ref[...][:, 0:1]
  z = s + agg * rd + b_ref[...]
  col = lax.broadcasted_iota(jnp.int32, z.shape, 1)
  valid = col < C
  zm = jnp.where(valid, z, -jnp.inf)
  m = jnp.max(zm, axis=1, keepdims=True)
  ex = jnp.where(valid, jnp.exp(zm - m), 0.0)
  lse = jnp.log(jnp.sum(ex, axis=1, keepdims=True))
  o_ref[...] = zm - m - lse


def _tc_final(z2, st2, g, be, Ws, aggp, rdeg, b):
  Win = z2.shape[1]
  W = Ws.shape[1]
  grid = (N // R_BLK,)
  return pl.pallas_call(
      _k_final_body,
      grid=grid,
      in_specs=[
          pl.BlockSpec((R_BLK, Win), lambda i: (i, 0)),
          pl.BlockSpec((8, Win), lambda i: (0, 0)),
          pl.BlockSpec((1, Win), lambda i: (0, 0)),
          pl.BlockSpec((1, Win), lambda i: (0, 0)),
          pl.BlockSpec((Win, W), lambda i: (0, 0)),
          pl.BlockSpec((NCORE, R_BLK, W), lambda i: (0, i, 0)),
          pl.BlockSpec((R_BLK, 16), lambda i: (i, 0)),
          pl.BlockSpec((1, W), lambda i: (0, 0)),
      ],
      out_specs=pl.BlockSpec((R_BLK, W), lambda i: (i, 0)),
      out_shape=jax.ShapeDtypeStruct((N, W), jnp.float32),
  )(z2, st2, g, be, Ws, aggp, rdeg, b)


def kernel(x, edge_index, W_self1, W_neigh1, b1, gamma1, beta1,
           W_self2, W_neigh2, b2, gamma2, beta2,
           W_self3, W_neigh3, b3):
  src = jnp.concatenate(
      [edge_index[0], jnp.zeros((EP - E,), jnp.int32)]).reshape(-1, CHUNK)
  pad_dst = N + jnp.arange(EP - E, dtype=jnp.int32) % (ACC_R - N)
  dst = jnp.concatenate([edge_index[1], pad_dst]).reshape(-1, CHUNK)
  zw128 = jnp.zeros((ZROWS, 128), jnp.float32)
  ones = jnp.ones((CHUNK, 128), jnp.float32)

  b1r = b1.reshape(1, H)
  b2r = b2.reshape(1, H)
  g1 = gamma1.reshape(1, H)
  be1 = beta1.reshape(1, H)
  g2 = gamma2.reshape(1, H)
  be2 = beta2.reshape(1, H)
  Wn3 = jnp.pad(W_neigh3, ((0, 0), (0, 128 - C)))
  Ws3 = jnp.pad(W_self3, ((0, 0), (0, 128 - C)))
  b3r = jnp.pad(b3, (0, 128 - C)).reshape(1, 128)

  degp = _sc_degree(dst, ones, zw128)
  p1lo, p1hi = _tc_proj(x, W_neigh1, split_out=True)
  agg1 = _sc_segment_sum(p1lo, p1hi, src, dst, zw128, edge_split=False)
  z1, st1, rdeg = _tc_combine_stats(x, None, W_self1, agg1, degp, b1r,
                                    deg_partials=True)
  p2lo, p2hi = _tc_bn_proj(z1, st1, g1, be1, W_neigh2, split_out=True)
  agg2 = _sc_segment_sum(p2lo, p2hi, src, dst, zw128, edge_split=False)
  z2, st2 = _tc_combine_stats(z1, (st1, g1, be1), W_self2, agg2, rdeg, b2r,
                              deg_partials=False)
  p3 = _tc_bn_proj(z2, st2, g2, be2, Wn3, split_out=False)
  agg3 = _sc_segment_sum(p3, p3, src, dst, zw128, edge_split=True)
  o = _tc_final(z2, st2, g2, be2, Ws3, agg3, rdeg, b3r)
  return o[:, :C]


# speedup vs baseline: 1.0206x; 1.0027x over previous
import jax
import jax.numpy as jnp
from jax import lax
from jax.experimental import pallas as pl
from jax.experimental.pallas import tpu as pltpu
from jax.experimental.pallas import tpu_sc as plsc

N = 10000
D = 256
H = 256
C = 47
E = 160000

NCORE = 2
NSUB = 16
CHUNK = 128
EP = 163840
EPT_COL = EP // NSUB
NCH_COL = EPT_COL // CHUNK
EPT_EDGE = EP // (NSUB * NCORE)
NCH_EDGE = EPT_EDGE // CHUNK
ACC_R = 10240
ZROWS = ACC_R // NSUB
R_BLK = 5000
EPS = 1e-5


def _sc_copy_out(sid, src_ref, mk_dst, stripe=632):
  @pl.when(sid < NSUB - 1)
  def _():
    sl = pl.ds(sid * stripe, stripe)
    pltpu.sync_copy(src_ref.at[sl], mk_dst(sl))

  @pl.when(sid == NSUB - 1)
  def _():
    sl = pl.ds((NSUB - 1) * stripe, N - (NSUB - 1) * stripe)
    pltpu.sync_copy(src_ref.at[sl], mk_dst(sl))


def _sc_segment_sum(p_lo, p_hi, src2d, dst2d, zeros_w, *, edge_split):
  NCH_E0, NCH_E1 = 64, 16
  nch = max(NCH_E0, NCH_E1) if edge_split else NCH_COL
  nhalf = 1 if edge_split else 2
  nch2 = nch // nhalf
  stripe = 632
  mesh = plsc.VectorSubcoreMesh(core_axis_name="c", subcore_axis_name="s")
  out_type = [jax.ShapeDtypeStruct((NCORE, N, 128), jnp.float32)]
  scratch = [
      pltpu.VMEM((nch2, CHUNK), jnp.int32),
      pltpu.VMEM((nch2, CHUNK), jnp.int32),
      pltpu.VMEM((CHUNK, 128), jnp.float32),
      pltpu.VMEM((CHUNK, 128), jnp.float32),
      pltpu.VMEM_SHARED((ACC_R, 128), jnp.float32),
      pltpu.SemaphoreType.DMA,
      pltpu.SemaphoreType.DMA,
  ]

  def body(plo_hbm, phi_hbm, src_hbm, dst_hbm, zw_hbm, agg_hbm, sidx, didx,
           rows0, rows1, acc, sem0, sem1):
    core = lax.axis_index("c")
    sid = lax.axis_index("s")

    pltpu.sync_copy(zw_hbm, acc.at[pl.ds(sid * ZROWS, ZROWS)])
    plsc.subcore_barrier()

    def run(tbl, crow0, my_nch):
      for h in range(nhalf if my_nch == nch2 * nhalf else 1):
        rows_here = my_nch if my_nch < nch2 * nhalf else nch2
        pltpu.sync_copy(src_hbm.at[pl.ds(crow0 + h * rows_here, rows_here)],
                        sidx.at[pl.ds(0, rows_here)])
        pltpu.sync_copy(dst_hbm.at[pl.ds(crow0 + h * rows_here, rows_here)],
                        didx.at[pl.ds(0, rows_here)])
        pltpu.async_copy(tbl.at[sidx.at[0]], rows0, sem0)

        def step(k, carry):
          ca = 2 * k
          cb = 2 * k + 1
          pltpu.async_copy(tbl.at[sidx.at[cb]], rows1, sem1)
          pltpu.make_async_copy(tbl.at[sidx.at[ca]], rows0, sem0).wait()
          pltpu.sync_copy(rows0, acc.at[didx.at[ca]], add=True)

          @pl.when(k < rows_here // 2 - 1)
          def _():
            pltpu.async_copy(tbl.at[sidx.at[ca + 2]], rows0, sem0)
          pltpu.make_async_copy(tbl.at[sidx.at[cb]], rows1, sem1).wait()
          pltpu.sync_copy(rows1, acc.at[didx.at[cb]], add=True)
          return carry
        lax.fori_loop(0, rows_here // 2, step, 0)

    if edge_split:
      @pl.when(core == 0)
      def _():
        run(plo_hbm, sid * NCH_E0, NCH_E0)

      @pl.when(core == 1)
      def _():
        run(plo_hbm, NSUB * NCH_E0 + sid * NCH_E1, NCH_E1)
    else:
      @pl.when(core == 0)
      def _():
        run(plo_hbm, sid * NCH_COL, NCH_COL)

      @pl.when(core == 1)
      def _():
        run(phi_hbm, sid * NCH_COL, NCH_COL)

    plsc.subcore_barrier()

    @pl.when(core == 0)
    def _():
      _sc_copy_out(sid, acc, lambda sl: agg_hbm.at[0, sl], stripe)

    @pl.when(core == 1)
    def _():
      _sc_copy_out(sid, acc, lambda sl: agg_hbm.at[1, sl], stripe)

  fn = pl.kernel(body, mesh=mesh, out_type=out_type, scratch_types=scratch)
  res = fn(p_lo, p_hi, src2d, dst2d, zeros_w)
  return res[0] if isinstance(res, (list, tuple)) else res


def _sc_degree(dst2d, ones, zeros_w):
  mesh = plsc.VectorSubcoreMesh(core_axis_name="c", subcore_axis_name="s")
  out_type = [jax.ShapeDtypeStruct((NCORE, N, 128), jnp.float32)]
  scratch = [
      pltpu.VMEM((NCH_EDGE, CHUNK), jnp.int32),
      pltpu.VMEM((CHUNK, 128), jnp.float32),
      pltpu.VMEM_SHARED((ACC_R, 128), jnp.float32),
  ]

  def body(dst_hbm, ones_hbm, zw_hbm, deg_hbm, didx, onev, dacc):
    core = lax.axis_index("c")
    sid = lax.axis_index("s")
    pltpu.sync_copy(zw_hbm, dacc.at[pl.ds(sid * ZROWS, ZROWS)])
    pltpu.sync_copy(ones_hbm, onev)
    crow0 = (core * NSUB + sid) * NCH_EDGE
    pltpu.sync_copy(dst_hbm.at[pl.ds(crow0, NCH_EDGE)], didx)
    plsc.subcore_barrier()

    def step(ci, carry):
      pltpu.sync_copy(onev, dacc.at[didx.at[ci]], add=True)
      return carry
    lax.fori_loop(0, NCH_EDGE, step, 0)

    plsc.subcore_barrier()

    @pl.when(core == 0)
    def _():
      _sc_copy_out(sid, dacc, lambda sl: deg_hbm.at[0, sl])

    @pl.when(core == 1)
    def _():
      _sc_copy_out(sid, dacc, lambda sl: deg_hbm.at[1, sl])

  fn = pl.kernel(body, mesh=mesh, out_type=out_type, scratch_types=scratch)
  res = fn(dst2d, ones, zeros_w)
  return res[0] if isinstance(res, (list, tuple)) else res


def _make_proj_body(split_out):
  def body(x_ref, w_ref, *o_refs):
    r = jnp.dot(x_ref[...], w_ref[...], preferred_element_type=jnp.float32)
    if split_out:
      o_refs[0][...] = r[:, :128]
      o_refs[1][...] = r[:, 128:]
    else:
      o_refs[0][...] = r
  return body


def _proj_out(Wout, split_out):
  if split_out:
    return ([pl.BlockSpec((R_BLK, 128), lambda i: (i, 0)),
             pl.BlockSpec((R_BLK, 128), lambda i: (i, 0))],
            [jax.ShapeDtypeStruct((N, 128), jnp.float32),
             jax.ShapeDtypeStruct((N, 128), jnp.float32)])
  return (pl.BlockSpec((R_BLK, Wout), lambda i: (i, 0)),
          jax.ShapeDtypeStruct((N, Wout), jnp.float32))


def _tc_proj(x, W, *, split_out):
  Win = x.shape[1]
  Wout = W.shape[1]
  o_spec, o_shape = _proj_out(Wout, split_out)
  return pl.pallas_call(
      _make_proj_body(split_out),
      grid=(N // R_BLK,),
      in_specs=[
          pl.BlockSpec((R_BLK, Win), lambda i: (i, 0)),
          pl.BlockSpec((Win, Wout), lambda i: (0, 0)),
      ],
      out_specs=o_spec,
      out_shape=o_shape,
  )(x, W)


def _bn_relu(z, st, g, be):
  mean = st[0:1, :] / N
  var = st[1:2, :] / N - mean * mean
  inv = lax.rsqrt(var + EPS)
  return jnp.maximum((z - mean) * inv * g + be, 0.0)


def _make_combine_body(deg_partials, with_bn):
  def body(h_ref, *refs):
    if with_bn:
      pst_ref, g_ref, be_ref, w_ref, a_ref, d_ref, b_ref = refs[:7]
      out_refs = refs[7:]
      h = _bn_relu(h_ref[...], pst_ref[...], g_ref[...], be_ref[...])
    else:
      w_ref, a_ref, d_ref, b_ref = refs[:4]
      out_refs = refs[4:]
      h = h_ref[...]
    if deg_partials:
      z_ref, st_ref, rd_ref = out_refs
      d = d_ref[0][:, 0:1] + d_ref[1][:, 0:1]
      rd = 1.0 / jnp.maximum(d, 1.0)
      rd_ref[...] = jnp.broadcast_to(rd, (rd.shape[0], 16))
    else:
      z_ref, st_ref = out_refs
      rd = d_ref[...][:, 0:1]
    i = pl.program_id(0)
    s = jnp.dot(h, w_ref[...], preferred_element_type=jnp.float32)
    agg = jnp.concatenate([a_ref[0], a_ref[1]], axis=1)
    z = s + agg * rd + b_ref[...]
    z_ref[...] = z

    @pl.when(i == 0)
    def _():
      st_ref[...] = jnp.zeros_like(st_ref)

    w = z.shape[1]
    contrib = jnp.concatenate([
        jnp.sum(z, axis=0, keepdims=True),
        jnp.sum(z * z, axis=0, keepdims=True),
        jnp.zeros((6, w), jnp.float32),
    ], axis=0)
    st_ref[...] += contrib
  return body


def _tc_combine_stats(h, bn, Ws, aggp, d, b, *, deg_partials):
  Win = h.shape[1]
  W = Ws.shape[1]
  grid = (N // R_BLK,)
  if deg_partials:
    d_spec = pl.BlockSpec((NCORE, R_BLK, 128), lambda i: (0, i, 0))
  else:
    d_spec = pl.BlockSpec((R_BLK, 16), lambda i: (i, 0))
  in_specs = [pl.BlockSpec((R_BLK, Win), lambda i: (i, 0))]
  args = [h]
  if bn is not None:
    pst, g, be = bn
    in_specs += [
        pl.BlockSpec((8, Win), lambda i: (0, 0)),
        pl.BlockSpec((1, Win), lambda i: (0, 0)),
        pl.BlockSpec((1, Win), lambda i: (0, 0)),
    ]
    args += [pst, g, be]
  in_specs += [
      pl.BlockSpec((Win, W), lambda i: (0, 0)),
      pl.BlockSpec((NCORE, R_BLK, W // 2), lambda i: (0, i, 0)),
      d_spec,
      pl.BlockSpec((1, W), lambda i: (0, 0)),
  ]
  args += [Ws, aggp, d, b]
  out_specs = [
      pl.BlockSpec((R_BLK, W), lambda i: (i, 0)),
      pl.BlockSpec((8, W), lambda i: (0, 0)),
  ]
  out_shape = [
      jax.ShapeDtypeStruct((N, W), jnp.float32),
      jax.ShapeDtypeStruct((8, W), jnp.float32),
  ]
  if deg_partials:
    out_specs.append(pl.BlockSpec((R_BLK, 16), lambda i: (i, 0)))
    out_shape.append(jax.ShapeDtypeStruct((N, 16), jnp.float32))
  return pl.pallas_call(
      _make_combine_body(deg_partials, bn is not None),
      grid=grid,
      in_specs=in_specs,
      out_specs=out_specs,
      out_shape=out_shape,
  )(*args)


def _make_bn_proj_body(split_out):
  def body(z_ref, st_ref, g_ref, be_ref, w_ref, *o_refs):
    h = _bn_relu(z_ref[...], st_ref[...], g_ref[...], be_ref[...])
    r = jnp.dot(h, w_ref[...], preferred_element_type=jnp.float32)
    if split_out:
      o_refs[0][...] = r[:, :128]
      o_refs[1][...] = r[:, 128:]
    else:
      o_refs[0][...] = r
  return body


def _tc_bn_proj(z, st, g, be, W, *, split_out):
  Win = z.shape[1]
  Wout = W.shape[1]
  o_spec, o_shape = _proj_out(Wout, split_out)
  return pl.pallas_call(
      _make_bn_proj_body(split_out),
      grid=(N // R_BLK,),
      in_specs=[
          pl.BlockSpec((R_BLK, Win), lambda i: (i, 0)),
          pl.BlockSpec((8, Win), lambda i: (0, 0)),
          pl.BlockSpec((1, Win), lambda i: (0, 0)),
          pl.BlockSpec((1, Win), lambda i: (0, 0)),
          pl.BlockSpec((Win, Wout), lambda i: (0, 0)),
      ],
      out_specs=o_spec,
      out_shape=o_shape,
  )(z, st, g, be, W)


def _k_final_body(z2_ref, pst_ref, g_ref, be_ref, w_ref, a_ref, rd_ref, b_ref,
                  o_ref):
  h = _bn_relu(z2_ref[...], pst_ref[...], g_ref[...], be_ref[...])
  s = jnp.dot(h, w_ref[...], preferred_element_type=jnp.float32)
  agg = a_ref[0] + a_ref[1]
  rd = rd_ref[...][:, 0:1]
  z = s + agg * rd + b_ref[...]
  col = lax.broadcasted_iota(jnp.int32, z.shape, 1)
  valid = col < C
  zm = jnp.where(valid, z, -jnp.inf)
  m = jnp.max(zm, axis=1, keepdims=True)
  ex = jnp.where(valid, jnp.exp(zm - m), 0.0)
  lse = jnp.log(jnp.sum(ex, axis=1, keepdims=True))
  o_ref[...] = zm - m - lse


def _tc_final(z2, st2, g, be, Ws, aggp, rdeg, b):
  Win = z2.shape[1]
  W = Ws.shape[1]
  grid = (N // R_BLK,)
  return pl.pallas_call(
      _k_final_body,
      grid=grid,
      in_specs=[
          pl.BlockSpec((R_BLK, Win), lambda i: (i, 0)),
          pl.BlockSpec((8, Win), lambda i: (0, 0)),
          pl.BlockSpec((1, Win), lambda i: (0, 0)),
          pl.BlockSpec((1, Win), lambda i: (0, 0)),
          pl.BlockSpec((Win, W), lambda i: (0, 0)),
          pl.BlockSpec((NCORE, R_BLK, W), lambda i: (0, i, 0)),
          pl.BlockSpec((R_BLK, 16), lambda i: (i, 0)),
          pl.BlockSpec((1, W), lambda i: (0, 0)),
      ],
      out_specs=pl.BlockSpec((R_BLK, W), lambda i: (i, 0)),
      out_shape=jax.ShapeDtypeStruct((N, W), jnp.float32),
  )(z2, st2, g, be, Ws, aggp, rdeg, b)


def kernel(x, edge_index, W_self1, W_neigh1, b1, gamma1, beta1,
           W_self2, W_neigh2, b2, gamma2, beta2,
           W_self3, W_neigh3, b3):
  src = jnp.concatenate(
      [edge_index[0], jnp.zeros((EP - E,), jnp.int32)]).reshape(-1, CHUNK)
  pad_dst = N + jnp.arange(EP - E, dtype=jnp.int32) % (ACC_R - N)
  dst = jnp.concatenate([edge_index[1], pad_dst]).reshape(-1, CHUNK)
  zw128 = jnp.zeros((ZROWS, 128), jnp.float32)
  ones = jnp.ones((CHUNK, 128), jnp.float32)

  b1r = b1.reshape(1, H)
  b2r = b2.reshape(1, H)
  g1 = gamma1.reshape(1, H)
  be1 = beta1.reshape(1, H)
  g2 = gamma2.reshape(1, H)
  be2 = beta2.reshape(1, H)
  Wn3 = jnp.pad(W_neigh3, ((0, 0), (0, 128 - C)))
  Ws3 = jnp.pad(W_self3, ((0, 0), (0, 128 - C)))
  b3r = jnp.pad(b3, (0, 128 - C)).reshape(1, 128)

  degp = _sc_degree(dst, ones, zw128)
  p1lo, p1hi = _tc_proj(x, W_neigh1, split_out=True)
  agg1 = _sc_segment_sum(p1lo, p1hi, src, dst, zw128, edge_split=False)
  z1, st1, rdeg = _tc_combine_stats(x, None, W_self1, agg1, degp, b1r,
                                    deg_partials=True)
  p2lo, p2hi = _tc_bn_proj(z1, st1, g1, be1, W_neigh2, split_out=True)
  agg2 = _sc_segment_sum(p2lo, p2hi, src, dst, zw128, edge_split=False)
  z2, st2 = _tc_combine_stats(z1, (st1, g1, be1), W_self2, agg2, rdeg, b2r,
                              deg_partials=False)
  p3 = _tc_bn_proj(z2, st2, g2, be2, Wn3, split_out=False)
  agg3 = _sc_segment_sum(p3, p3, src, dst, zw128, edge_split=True)
  o = _tc_final(z2, st2, g2, be2, Ws3, agg3, rdeg, b3r)
  return o[:, :C]
